# Initial kernel scaffold; baseline (speedup 1.0000x reference)
import jax
import jax.numpy as jnp
from jax.experimental import pallas as pl


def kernel(probs, prev_probs):
    raise NotImplementedError("write your pallas kernel here")


# speedup vs baseline: 4.6602x; 4.6602x over previous
import functools

import jax
import jax.numpy as jnp
from jax.experimental import pallas as pl
from jax.experimental.pallas import tpu as pltpu

_TOP_K = 256
_ALPHA = 0.05
_PEN = 0.15
_N = 32768
_ROWS = 256
_COLS = 128


def _order_key(x):
    i = jax.lax.bitcast_convert_type(x, jnp.int32)
    return jnp.where(i >= 0, i, i ^ jnp.int32(0x7FFFFFFF))


def _tc_body(p_ref, q_ref, adj_ref, pen_ref):
    p = p_ref[...]
    q = q_ref[...]
    kp = _order_key(p)
    kq = _order_key(q)
    row = jax.lax.broadcasted_iota(jnp.int32, (_ROWS, _COLS), 0)
    col = jax.lax.broadcasted_iota(jnp.int32, (_ROWS, _COLS), 1)
    idx = row * _COLS + col

    k = jnp.int32(_TOP_K)

    def cnt_ge(keys, t):
        return jnp.sum((keys >= t).astype(jnp.int32))

    int_min = jnp.int32(-2147483648)
    base_p = jnp.where(cnt_ge(kp, 0) >= k, jnp.int32(0), int_min)
    base_q = jnp.where(cnt_ge(kq, 0) >= k, jnp.int32(0), int_min)

    def val_bit(i, carry):
        m_p, m_q = carry
        bit = jnp.int32(1) << (jnp.int32(30) - i)
        c_p = base_p + (m_p | bit)
        c_q = base_q + (m_q | bit)
        m_p = jnp.where(cnt_ge(kp, c_p) >= k, m_p | bit, m_p)
        m_q = jnp.where(cnt_ge(kq, c_q) >= k, m_q | bit, m_q)
        return m_p, m_q

    m_p, m_q = jax.lax.fori_loop(0, 31, val_bit, (jnp.int32(0), jnp.int32(0)))
    t_p = base_p + m_p
    t_q = base_q + m_q

    eq_p = kp == t_p
    eq_q = kq == t_q
    r_p = k - jnp.sum((kp > t_p).astype(jnp.int32))
    r_q = k - jnp.sum((kq > t_q).astype(jnp.int32))

    def cnt_eq_below(eq, m):
        return jnp.sum((eq & (idx < m)).astype(jnp.int32))

    def idx_bit(i, carry):
        mm_p, mm_q = carry
        bit = jnp.int32(1) << (jnp.int32(15) - i)
        c_p = mm_p | bit
        c_q = mm_q | bit
        mm_p = jnp.where(cnt_eq_below(eq_p, c_p) <= r_p, c_p, mm_p)
        mm_q = jnp.where(cnt_eq_below(eq_q, c_q) <= r_q, c_q, mm_q)
        return mm_p, mm_q

    big_m_p, big_m_q = jax.lax.fori_loop(
        0, 16, idx_bit, (jnp.int32(0), jnp.int32(0)))

    curr = (kp > t_p) | (eq_p & (idx < big_m_p))
    prev = (kq > t_q) | (eq_q & (idx < big_m_q))

    zero = jnp.float32(0.0)
    upper = jnp.sum(jnp.where(curr & (~prev), p, zero))
    lower = jnp.sum(jnp.where(prev & (~curr), q, zero))

    blend = p * jnp.float32(1.0 - _ALPHA) + q * jnp.float32(_ALPHA)
    s = jnp.sum(blend)
    adj_ref[...] = blend / (s + jnp.float32(1e-12))
    pen_ref[0, 0] = jnp.float32(_PEN) * (upper + lower)


@functools.partial(jax.jit)
def kernel(probs, prev_probs):
    p2 = probs.reshape(_ROWS, _COLS)
    q2 = prev_probs.reshape(_ROWS, _COLS)
    adj, pen = pl.pallas_call(
        _tc_body,
        out_shape=(
            jax.ShapeDtypeStruct((_ROWS, _COLS), jnp.float32),
            jax.ShapeDtypeStruct((1, 1), jnp.float32),
        ),
        out_specs=(
            pl.BlockSpec(memory_space=pltpu.VMEM),
            pl.BlockSpec(memory_space=pltpu.SMEM),
        ),
    )(p2, q2)
    return adj.reshape(_N), jax.lax.stop_gradient(pen[0, 0])
